# Initial kernel scaffold; baseline (speedup 1.0000x reference)
import jax
import jax.numpy as jnp
from jax.experimental import pallas as pl


def kernel(observations, edge_index, Wg_z, bg_z, Wl_z, bl_z, Wg_r, bg_r, Wl_r, bl_r, Wg_h, bg_h, Wl_h, bl_h):
    raise NotImplementedError("write your pallas kernel here")


# speedup vs baseline: 185.3094x; 185.3094x over previous
import functools

import jax
import jax.numpy as jnp
from jax import lax
from jax.experimental import pallas as pl
from jax.experimental.pallas import tpu as pltpu
from jax.experimental.pallas import tpu_sc as plsc

N = 100000
E = 3200000
HID = 16
NC = 2
NS = 16
NW = NC * NS
N_PAD = 102400
SLICE = N_PAD // NS
EPW = E // NW
CH = 20000
NCHUNK = EPW // CH

_mesh = plsc.VectorSubcoreMesh(core_axis_name="c", subcore_axis_name="s")


def _fill(ref, value, n):
  vec = jnp.full((16,), value, jnp.float32)
  def body(i, _):
    ref[pl.ds(i * 16, 16)] = vec
    return ()
  lax.fori_loop(0, n // 16, body, ())


def _hist_body(col_hbm, out_hbm, acc, colv, ones_v, zv):
  cid = lax.axis_index("c")
  sid = lax.axis_index("s")
  wid = cid * NS + sid
  _fill(ones_v, 1.0, CH)
  _fill(zv, 0.0, SLICE)
  pltpu.sync_copy(zv, acc.at[pl.ds(sid * SLICE, SLICE)])
  plsc.subcore_barrier()

  def chunk(c, _):
    base = wid * EPW + c * CH
    pltpu.sync_copy(col_hbm.at[pl.ds(base, CH)], colv)
    pltpu.sync_copy(ones_v, acc.at[colv], add=True)
    return ()
  lax.fori_loop(0, NCHUNK, chunk, ())

  plsc.subcore_barrier()
  pltpu.sync_copy(acc.at[pl.ds(sid * SLICE, SLICE)],
                  out_hbm.at[cid, pl.ds(sid * SLICE, SLICE)])


_hist = functools.partial(
    pl.kernel,
    out_type=jax.ShapeDtypeStruct((NC, N_PAD), jnp.float32),
    mesh=_mesh,
    scratch_types=[
        pltpu.VMEM_SHARED((N_PAD,), jnp.float32),
        pltpu.VMEM((CH,), jnp.int32),
        pltpu.VMEM((CH,), jnp.float32),
        pltpu.VMEM((SLICE,), jnp.float32),
    ],
)(_hist_body)


def _gs_body(row_hbm, col_hbm, p_hbm, out_hbm, p_sp, acc, rowv, colv, vals,
             zv, sem):
  cid = lax.axis_index("c")
  sid = lax.axis_index("s")
  wid = cid * NS + sid
  _fill(zv, 0.0, SLICE)
  pltpu.sync_copy(zv, acc.at[pl.ds(sid * SLICE, SLICE)])
  pltpu.sync_copy(p_hbm.at[pl.ds(sid * SLICE, SLICE)],
                  p_sp.at[pl.ds(sid * SLICE, SLICE)])
  plsc.subcore_barrier()

  def chunk(c, _):
    base = wid * EPW + c * CH
    pltpu.sync_copy(row_hbm.at[pl.ds(base, CH)], rowv)
    pltpu.sync_copy(col_hbm.at[pl.ds(base, CH)], colv)
    pltpu.async_copy(p_sp.at[rowv], vals, sem).wait()
    pltpu.sync_copy(vals, acc.at[colv], add=True)
    return ()
  lax.fori_loop(0, NCHUNK, chunk, ())

  plsc.subcore_barrier()
  pltpu.sync_copy(acc.at[pl.ds(sid * SLICE, SLICE)],
                  out_hbm.at[cid, pl.ds(sid * SLICE, SLICE)])


_gs = functools.partial(
    pl.kernel,
    out_type=jax.ShapeDtypeStruct((NC, N_PAD), jnp.float32),
    mesh=_mesh,
    scratch_types=[
        pltpu.VMEM_SHARED((N_PAD,), jnp.float32),
        pltpu.VMEM_SHARED((N_PAD,), jnp.float32),
        pltpu.VMEM((CH,), jnp.int32),
        pltpu.VMEM((CH,), jnp.int32),
        pltpu.VMEM((CH,), jnp.float32),
        pltpu.VMEM((SLICE,), jnp.float32),
        pltpu.SemaphoreType.DMA,
    ],
)(_gs_body)


BN = 5120


def _prep_body(deg2_ref, obs_ref, dinv_ref, p_ref):
  deg = deg2_ref[0, :] + deg2_ref[1, :] + 1.0
  dinv = lax.rsqrt(deg)
  dinv_ref[...] = dinv
  p_ref[...] = dinv * obs_ref[:, 4]


def _prep(deg2, obs_pad):
  return pl.pallas_call(
      _prep_body,
      grid=(N_PAD // BN,),
      in_specs=[
          pl.BlockSpec((NC, BN), lambda i: (0, i)),
          pl.BlockSpec((BN, 5), lambda i: (i, 0)),
      ],
      out_specs=[
          pl.BlockSpec((BN,), lambda i: (i,)),
          pl.BlockSpec((BN,), lambda i: (i,)),
      ],
      out_shape=[
          jax.ShapeDtypeStruct((N_PAD,), jnp.float32),
          jax.ShapeDtypeStruct((N_PAD,), jnp.float32),
      ],
  )(deg2, obs_pad)


def _fin_body(t2_ref, dinv_ref, obs_ref, Wgz_ref, bgz_ref, Wlz_ref, blz_ref,
              Wgh_ref, bgh_ref, Wlh_ref, blh_ref, out_ref):
  Az = Wlz_ref[0:HID, :]
  uz = jnp.sum(Wgz_ref[0, :][:, None] * Az, axis=0)
  vz = jnp.sum(bgz_ref[:][:, None] * Az, axis=0) + blz_ref[...]
  Ah = Wlh_ref[0:HID, :]
  uh = jnp.sum(Wgh_ref[0, :][:, None] * Ah, axis=0)
  vh = jnp.sum(bgh_ref[:][:, None] * Ah, axis=0) + blh_ref[...]
  t = t2_ref[0, :] + t2_ref[1, :]
  dinv = dinv_ref[...]
  x = obs_ref[:, 4]
  agg = dinv * t + dinv * dinv * x
  Z = jax.nn.sigmoid(agg[:, None] * uz[None, :] + vz[None, :])
  Ht = jnp.tanh(agg[:, None] * uh[None, :] + vh[None, :])
  H = (1.0 - Z) * Ht
  out_ref[...] = jnp.concatenate([obs_ref[:, 0:4], H], axis=1)[None]


def _fin(t2, dinv, obs_pad, Wg_z, bg_z, Wl_z, bl_z, Wg_h, bg_h, Wl_h, bl_h):
  def full(shape):
    return pl.BlockSpec(shape, lambda i, _s=shape: tuple(0 for _ in _s))
  return pl.pallas_call(
      _fin_body,
      grid=(N_PAD // BN,),
      in_specs=[
          pl.BlockSpec((NC, BN), lambda i: (0, i)),
          pl.BlockSpec((BN,), lambda i: (i,)),
          pl.BlockSpec((BN, 5), lambda i: (i, 0)),
          full((1, HID)), full((HID,)), full((2 * HID, HID)), full((HID,)),
          full((1, HID)), full((HID,)), full((2 * HID, HID)), full((HID,)),
      ],
      out_specs=pl.BlockSpec((1, BN, 4 + HID), lambda i: (0, i, 0)),
      out_shape=jax.ShapeDtypeStruct((1, N_PAD, 4 + HID), jnp.float32),
  )(t2, dinv, obs_pad, Wg_z, bg_z, Wl_z, bl_z, Wg_h, bg_h, Wl_h, bl_h)


def kernel(observations, edge_index, Wg_z, bg_z, Wl_z, bl_z, Wg_r, bg_r,
           Wl_r, bl_r, Wg_h, bg_h, Wl_h, bl_h):
  del Wg_r, bg_r, Wl_r, bl_r
  row = edge_index[0]
  col = edge_index[1]
  obs_pad = jnp.pad(observations, ((0, N_PAD - N), (0, 0)))
  deg2 = _hist(col)
  dinv, p = _prep(deg2, obs_pad)
  t2 = _gs(row, col, p)
  out_pad = _fin(t2, dinv, obs_pad, Wg_z, bg_z, Wl_z, bl_z,
                 Wg_h, bg_h, Wl_h, bl_h)
  return out_pad[:, :N, :]


# speedup vs baseline: 383.0933x; 2.0673x over previous
import functools

import jax
import jax.numpy as jnp
from jax import lax
from jax.experimental import pallas as pl
from jax.experimental.pallas import tpu as pltpu
from jax.experimental.pallas import tpu_sc as plsc

N = 100000
E = 3200000
HID = 16
NC = 2
NS = 16
NW = NC * NS
N_PAD = 102400
SLICE = N_PAD // NS
CH = 5120
NCHUNK = E // CH
FULL_ROUNDS = NCHUNK // NW
TAIL_BASE = FULL_ROUNDS * NW

_mesh = plsc.VectorSubcoreMesh(core_axis_name="c", subcore_axis_name="s")


def _fill(ref, value, n):
  vec = jnp.full((16,), value, jnp.float32)
  def body(i, _):
    ref[pl.ds(i * 16, 16)] = vec
    return ()
  lax.fori_loop(0, n // 16, body, ())


def _hist_body(ei_hbm, outa_hbm, outb_hbm, acc, eis, colv0, colv1, ones_v,
               zv, sem_ei0, sem_ei1, sem_c0, sem_c1, sem_s0, sem_s1):
  cid = lax.axis_index("c")
  sid = lax.axis_index("s")
  wid = cid * NS + sid
  _fill(ones_v, 1.0, CH)
  _fill(zv, 0.0, SLICE)
  pltpu.sync_copy(zv, acc.at[pl.ds(sid * SLICE, SLICE)])
  plsc.subcore_barrier()

  colv = (colv0, colv1)
  sem_ei = (sem_ei0, sem_ei1)
  sem_c = (sem_c0, sem_c1)
  sem_s = (sem_s0, sem_s1)
  ei_d, c_d, s_d = {}, {}, {}

  def start_ei(k):
    b = k % 2
    ei_d[k] = pltpu.async_copy(
        ei_hbm.at[:, pl.ds((k * NW + wid) * CH, CH)], eis.at[sid, b],
        sem_ei[b])

  start_ei(0)
  for k in range(FULL_ROUNDS):
    b = k % 2
    if k >= 2:
      s_d[k - 2].wait()
    ei_d[k].wait()
    c_d[k] = pltpu.async_copy(eis.at[sid, b, 1], colv[b], sem_c[b])
    if k + 1 < FULL_ROUNDS:
      start_ei(k + 1)
    c_d[k].wait()
    s_d[k] = pltpu.async_copy(ones_v, acc.at[colv[b]], sem_s[b], add=True)
  for k in range(max(FULL_ROUNDS - 2, 0), FULL_ROUNDS):
    s_d[k].wait()

  @pl.when(TAIL_BASE + wid < NCHUNK)
  def _():
    pltpu.sync_copy(ei_hbm.at[:, pl.ds((TAIL_BASE + wid) * CH, CH)],
                    eis.at[sid, 0])
    pltpu.sync_copy(eis.at[sid, 0, 1], colv0)
    pltpu.sync_copy(ones_v, acc.at[colv0], add=True)

  plsc.subcore_barrier()
  sl = pl.ds(sid * SLICE, SLICE)
  @pl.when(cid == 0)
  def _():
    pltpu.sync_copy(acc.at[sl], outa_hbm.at[sl])
  @pl.when(cid == 1)
  def _():
    pltpu.sync_copy(acc.at[sl], outb_hbm.at[sl])


_hist = functools.partial(
    pl.kernel,
    out_type=[
        jax.ShapeDtypeStruct((N_PAD,), jnp.float32),
        jax.ShapeDtypeStruct((N_PAD,), jnp.float32),
    ],
    mesh=_mesh,
    scratch_types=[
        pltpu.VMEM_SHARED((N_PAD,), jnp.float32),
        pltpu.VMEM_SHARED((NS, 2, 2, CH), jnp.int32),
        pltpu.VMEM((CH,), jnp.int32),
        pltpu.VMEM((CH,), jnp.int32),
        pltpu.VMEM((CH,), jnp.float32),
        pltpu.VMEM((SLICE,), jnp.float32),
        pltpu.SemaphoreType.DMA,
        pltpu.SemaphoreType.DMA,
        pltpu.SemaphoreType.DMA,
        pltpu.SemaphoreType.DMA,
        pltpu.SemaphoreType.DMA,
        pltpu.SemaphoreType.DMA,
    ],
)(_hist_body)


def _gs_body(ei_hbm, p_hbm, ta_hbm, tb_hbm,
             p_sp, acc, eis, rowv0, rowv1, colv0, colv1, vals0, vals1, zv,
             sem_p,
             sem_ei0, sem_ei1, sem_r0, sem_r1, sem_c0, sem_c1,
             sem_g0, sem_g1, sem_s0, sem_s1):
  cid = lax.axis_index("c")
  sid = lax.axis_index("s")
  wid = cid * NS + sid
  sl = pl.ds(sid * SLICE, SLICE)

  dp = pltpu.async_copy(p_hbm.at[sl], p_sp.at[sl], sem_p)
  _fill(zv, 0.0, SLICE)
  pltpu.sync_copy(zv, acc.at[sl])
  dp.wait()
  plsc.subcore_barrier()

  rowv = (rowv0, rowv1)
  colv = (colv0, colv1)
  vals = (vals0, vals1)
  sem_ei = (sem_ei0, sem_ei1)
  sem_r = (sem_r0, sem_r1)
  sem_c = (sem_c0, sem_c1)
  sem_g = (sem_g0, sem_g1)
  sem_s = (sem_s0, sem_s1)
  ei_d, r_d, c_d, g_d, s_d = {}, {}, {}, {}, {}

  def start_ei(k):
    b = k % 2
    ei_d[k] = pltpu.async_copy(
        ei_hbm.at[:, pl.ds((k * NW + wid) * CH, CH)], eis.at[sid, b],
        sem_ei[b])

  start_ei(0)
  for k in range(FULL_ROUNDS):
    b = k % 2
    if k >= 2:
      s_d[k - 2].wait()
    ei_d[k].wait()
    r_d[k] = pltpu.async_copy(eis.at[sid, b, 0], rowv[b], sem_r[b])
    c_d[k] = pltpu.async_copy(eis.at[sid, b, 1], colv[b], sem_c[b])
    if k + 1 < FULL_ROUNDS:
      start_ei(k + 1)
    r_d[k].wait()
    g_d[k] = pltpu.async_copy(p_sp.at[rowv[b]], vals[b], sem_g[b])
    c_d[k].wait()
    g_d[k].wait()
    s_d[k] = pltpu.async_copy(vals[b], acc.at[colv[b]], sem_s[b], add=True)
  for k in range(max(FULL_ROUNDS - 2, 0), FULL_ROUNDS):
    s_d[k].wait()

  @pl.when(TAIL_BASE + wid < NCHUNK)
  def _():
    pltpu.sync_copy(ei_hbm.at[:, pl.ds((TAIL_BASE + wid) * CH, CH)],
                    eis.at[sid, 0])
    pltpu.sync_copy(eis.at[sid, 0, 0], rowv0)
    pltpu.sync_copy(eis.at[sid, 0, 1], colv0)
    pltpu.async_copy(p_sp.at[rowv0], vals0, sem_g0).wait()
    pltpu.sync_copy(vals0, acc.at[colv0], add=True)

  plsc.subcore_barrier()
  @pl.when(cid == 0)
  def _():
    pltpu.sync_copy(acc.at[sl], ta_hbm.at[sl])
  @pl.when(cid == 1)
  def _():
    pltpu.sync_copy(acc.at[sl], tb_hbm.at[sl])


_gs = functools.partial(
    pl.kernel,
    out_type=[
        jax.ShapeDtypeStruct((N_PAD,), jnp.float32),
        jax.ShapeDtypeStruct((N_PAD,), jnp.float32),
    ],
    mesh=_mesh,
    scratch_types=[
        pltpu.VMEM_SHARED((N_PAD,), jnp.float32),
        pltpu.VMEM_SHARED((N_PAD,), jnp.float32),
        pltpu.VMEM_SHARED((NS, 2, 2, CH), jnp.int32),
        pltpu.VMEM((CH,), jnp.int32),
        pltpu.VMEM((CH,), jnp.int32),
        pltpu.VMEM((CH,), jnp.int32),
        pltpu.VMEM((CH,), jnp.int32),
        pltpu.VMEM((CH,), jnp.float32),
        pltpu.VMEM((CH,), jnp.float32),
        pltpu.VMEM((SLICE,), jnp.float32),
    ] + [pltpu.SemaphoreType.DMA] * 11,
)(_gs_body)


BN = 5120


def _prep_body(dega_ref, degb_ref, xp_ref, p_ref):
  deg = dega_ref[...] + degb_ref[...] + 1.0
  p_ref[...] = lax.rsqrt(deg) * xp_ref[...]


def _prep(dega, degb, xp):
  vec = pl.BlockSpec((BN,), lambda i: (i,))
  return pl.pallas_call(
      _prep_body,
      grid=(N_PAD // BN,),
      in_specs=[vec, vec, vec],
      out_specs=vec,
      out_shape=jax.ShapeDtypeStruct((N_PAD,), jnp.float32),
  )(dega, degb, xp)


def _fin_body(dega_ref, degb_ref, ta_ref, tb_ref, p_ref, obs_ref,
              Wgz_ref, bgz_ref, Wlz_ref, blz_ref,
              Wgh_ref, bgh_ref, Wlh_ref, blh_ref, out_ref):
  Az = Wlz_ref[0:HID, :]
  uz = jnp.sum(Wgz_ref[0, :][:, None] * Az, axis=0)
  vz = jnp.sum(bgz_ref[:][:, None] * Az, axis=0) + blz_ref[...]
  Ah = Wlh_ref[0:HID, :]
  uh = jnp.sum(Wgh_ref[0, :][:, None] * Ah, axis=0)
  vh = jnp.sum(bgh_ref[:][:, None] * Ah, axis=0) + blh_ref[...]
  deg = dega_ref[...] + degb_ref[...] + 1.0
  dinv = lax.rsqrt(deg)
  t = ta_ref[...] + tb_ref[...]
  agg = dinv * t + dinv * p_ref[...]
  Z = jax.nn.sigmoid(agg[:, None] * uz[None, :] + vz[None, :])
  Ht = jnp.tanh(agg[:, None] * uh[None, :] + vh[None, :])
  H = (1.0 - Z) * Ht
  out_ref[...] = jnp.concatenate([obs_ref[:, 0:4], H], axis=1)[None]


def _fin(dega, degb, ta, tb, p, obs,
         Wg_z, bg_z, Wl_z, bl_z, Wg_h, bg_h, Wl_h, bl_h):
  def full(shape):
    return pl.BlockSpec(shape, lambda i, _s=shape: tuple(0 for _ in _s))
  vec = pl.BlockSpec((BN,), lambda i: (i,))
  return pl.pallas_call(
      _fin_body,
      grid=(N_PAD // BN,),
      in_specs=[
          vec, vec, vec, vec, vec,
          pl.BlockSpec((BN, 5), lambda i: (i, 0)),
          full((1, HID)), full((HID,)), full((2 * HID, HID)), full((HID,)),
          full((1, HID)), full((HID,)), full((2 * HID, HID)), full((HID,)),
      ],
      out_specs=pl.BlockSpec((1, BN, 4 + HID), lambda i: (0, i, 0)),
      out_shape=jax.ShapeDtypeStruct((1, N, 4 + HID), jnp.float32),
  )(dega, degb, ta, tb, p, obs,
    Wg_z, bg_z, Wl_z, bl_z, Wg_h, bg_h, Wl_h, bl_h)


def kernel(observations, edge_index, Wg_z, bg_z, Wl_z, bl_z, Wg_r, bg_r,
           Wl_r, bl_r, Wg_h, bg_h, Wl_h, bl_h):
  del Wg_r, bg_r, Wl_r, bl_r
  xp = jnp.pad(observations[:, 4], (0, N_PAD - N))
  dega, degb = _hist(edge_index)
  p = _prep(dega, degb, xp)
  ta, tb = _gs(edge_index, p)
  return _fin(dega, degb, ta, tb, p, observations,
              Wg_z, bg_z, Wl_z, bl_z, Wg_h, bg_h, Wl_h, bl_h)


# speedup vs baseline: 384.0885x; 1.0026x over previous
import functools

import jax
import jax.numpy as jnp
from jax import lax
from jax.experimental import pallas as pl
from jax.experimental.pallas import tpu as pltpu
from jax.experimental.pallas import tpu_sc as plsc

N = 100000
E = 3200000
HID = 16
NC = 2
NS = 16
NW = NC * NS
N_PAD = 102400
SLICE = N_PAD // NS
CH = 5120
NCHUNK = E // CH
FULL_ROUNDS = NCHUNK // NW
TAIL_BASE = FULL_ROUNDS * NW

_mesh = plsc.VectorSubcoreMesh(core_axis_name="c", subcore_axis_name="s")


def _fill(ref, value, n):
  vec = jnp.full((16,), value, jnp.float32)
  def body(i, _):
    ref[pl.ds(i * 16, 16)] = vec
    return ()
  lax.fori_loop(0, n // 16, body, ())


def _hist_body(ei_hbm, outa_hbm, outb_hbm, acc, eis, colv0, colv1, colv2,
               ones_v, zv, sem_ei0, sem_ei1, sem_ei2, sem_c0, sem_c1, sem_c2,
               sem_s0, sem_s1, sem_s2):
  cid = lax.axis_index("c")
  sid = lax.axis_index("s")
  wid = cid * NS + sid
  _fill(ones_v, 1.0, CH)
  _fill(zv, 0.0, SLICE)
  pltpu.sync_copy(zv, acc.at[pl.ds(sid * SLICE, SLICE)])
  plsc.subcore_barrier()

  colv = (colv0, colv1, colv2)
  sem_ei = (sem_ei0, sem_ei1, sem_ei2)
  sem_c = (sem_c0, sem_c1, sem_c2)
  sem_s = (sem_s0, sem_s1, sem_s2)
  ei_d, c_d, s_d = {}, {}, {}
  R = FULL_ROUNDS

  def start_ei(k):
    b = k % 3
    ei_d[k] = pltpu.async_copy(
        ei_hbm.at[:, pl.ds((k * NW + wid) * CH, CH)], eis.at[sid, b],
        sem_ei[b])

  start_ei(0)
  start_ei(1)
  ei_d[0].wait()
  c_d[0] = pltpu.async_copy(eis.at[sid, 0, 1], colv[0], sem_c[0])
  for t in range(R):
    b = t % 3
    if t + 2 < R:
      start_ei(t + 2)
    c_d[t].wait()
    s_d[t] = pltpu.async_copy(ones_v, acc.at[colv[b]], sem_s[b], add=True)
    if t + 1 < R:
      if t >= 2:
        s_d[t - 2].wait()
      ei_d[t + 1].wait()
      b1 = (t + 1) % 3
      c_d[t + 1] = pltpu.async_copy(eis.at[sid, b1, 1], colv[b1], sem_c[b1])
  for t in range(max(R - 2, 0), R):
    s_d[t].wait()

  @pl.when(TAIL_BASE + wid < NCHUNK)
  def _():
    pltpu.sync_copy(ei_hbm.at[:, pl.ds((TAIL_BASE + wid) * CH, CH)],
                    eis.at[sid, 0])
    pltpu.sync_copy(eis.at[sid, 0, 1], colv0)
    pltpu.sync_copy(ones_v, acc.at[colv0], add=True)

  plsc.subcore_barrier()
  sl = pl.ds(sid * SLICE, SLICE)
  @pl.when(cid == 0)
  def _():
    pltpu.sync_copy(acc.at[sl], outa_hbm.at[sl])
  @pl.when(cid == 1)
  def _():
    pltpu.sync_copy(acc.at[sl], outb_hbm.at[sl])


_hist = functools.partial(
    pl.kernel,
    out_type=[
        jax.ShapeDtypeStruct((N_PAD,), jnp.float32),
        jax.ShapeDtypeStruct((N_PAD,), jnp.float32),
    ],
    mesh=_mesh,
    scratch_types=[
        pltpu.VMEM_SHARED((N_PAD,), jnp.float32),
        pltpu.VMEM_SHARED((NS, 3, 2, CH), jnp.int32),
        pltpu.VMEM((CH,), jnp.int32),
        pltpu.VMEM((CH,), jnp.int32),
        pltpu.VMEM((CH,), jnp.int32),
        pltpu.VMEM((CH,), jnp.float32),
        pltpu.VMEM((SLICE,), jnp.float32),
    ] + [pltpu.SemaphoreType.DMA] * 9,
)(_hist_body)


def _gs_body(ei_hbm, p_hbm, ta_hbm, tb_hbm,
             p_sp, acc, eis, rowv0, rowv1, rowv2, colv0, colv1, colv2,
             vals0, vals1, vals2, zv,
             sem_p,
             sem_ei0, sem_ei1, sem_ei2, sem_r0, sem_r1, sem_r2,
             sem_c0, sem_c1, sem_c2, sem_g0, sem_g1, sem_g2,
             sem_s0, sem_s1, sem_s2):
  cid = lax.axis_index("c")
  sid = lax.axis_index("s")
  wid = cid * NS + sid
  sl = pl.ds(sid * SLICE, SLICE)

  dp = pltpu.async_copy(p_hbm.at[sl], p_sp.at[sl], sem_p)
  _fill(zv, 0.0, SLICE)
  pltpu.sync_copy(zv, acc.at[sl])
  dp.wait()
  plsc.subcore_barrier()

  rowv = (rowv0, rowv1, rowv2)
  colv = (colv0, colv1, colv2)
  vals = (vals0, vals1, vals2)
  sem_ei = (sem_ei0, sem_ei1, sem_ei2)
  sem_r = (sem_r0, sem_r1, sem_r2)
  sem_c = (sem_c0, sem_c1, sem_c2)
  sem_g = (sem_g0, sem_g1, sem_g2)
  sem_s = (sem_s0, sem_s1, sem_s2)
  ei_d, r_d, c_d, g_d, s_d = {}, {}, {}, {}, {}
  R = FULL_ROUNDS

  def start_ei(k):
    b = k % 3
    ei_d[k] = pltpu.async_copy(
        ei_hbm.at[:, pl.ds((k * NW + wid) * CH, CH)], eis.at[sid, b],
        sem_ei[b])

  def start_rc(k):
    b = k % 3
    r_d[k] = pltpu.async_copy(eis.at[sid, b, 0], rowv[b], sem_r[b])
    c_d[k] = pltpu.async_copy(eis.at[sid, b, 1], colv[b], sem_c[b])

  start_ei(0)
  start_ei(1)
  ei_d[0].wait()
  start_rc(0)
  for t in range(R):
    b = t % 3
    if t + 2 < R:
      start_ei(t + 2)
    r_d[t].wait()
    g_d[t] = pltpu.async_copy(p_sp.at[rowv[b]], vals[b], sem_g[b])
    if t + 1 < R:
      if t >= 2:
        s_d[t - 2].wait()
      ei_d[t + 1].wait()
      start_rc(t + 1)
    g_d[t].wait()
    c_d[t].wait()
    s_d[t] = pltpu.async_copy(vals[b], acc.at[colv[b]], sem_s[b], add=True)
  for t in range(max(R - 2, 0), R):
    s_d[t].wait()

  @pl.when(TAIL_BASE + wid < NCHUNK)
  def _():
    pltpu.sync_copy(ei_hbm.at[:, pl.ds((TAIL_BASE + wid) * CH, CH)],
                    eis.at[sid, 0])
    pltpu.sync_copy(eis.at[sid, 0, 0], rowv0)
    pltpu.sync_copy(eis.at[sid, 0, 1], colv0)
    pltpu.async_copy(p_sp.at[rowv0], vals0, sem_g0).wait()
    pltpu.sync_copy(vals0, acc.at[colv0], add=True)

  plsc.subcore_barrier()
  @pl.when(cid == 0)
  def _():
    pltpu.sync_copy(acc.at[sl], ta_hbm.at[sl])
  @pl.when(cid == 1)
  def _():
    pltpu.sync_copy(acc.at[sl], tb_hbm.at[sl])


_gs = functools.partial(
    pl.kernel,
    out_type=[
        jax.ShapeDtypeStruct((N_PAD,), jnp.float32),
        jax.ShapeDtypeStruct((N_PAD,), jnp.float32),
    ],
    mesh=_mesh,
    scratch_types=[
        pltpu.VMEM_SHARED((N_PAD,), jnp.float32),
        pltpu.VMEM_SHARED((N_PAD,), jnp.float32),
        pltpu.VMEM_SHARED((NS, 3, 2, CH), jnp.int32),
        pltpu.VMEM((CH,), jnp.int32),
        pltpu.VMEM((CH,), jnp.int32),
        pltpu.VMEM((CH,), jnp.int32),
        pltpu.VMEM((CH,), jnp.int32),
        pltpu.VMEM((CH,), jnp.int32),
        pltpu.VMEM((CH,), jnp.int32),
        pltpu.VMEM((CH,), jnp.float32),
        pltpu.VMEM((CH,), jnp.float32),
        pltpu.VMEM((CH,), jnp.float32),
        pltpu.VMEM((SLICE,), jnp.float32),
    ] + [pltpu.SemaphoreType.DMA] * 16,
)(_gs_body)


BN = 5120


def _prep_body(dega_ref, degb_ref, xp_ref, p_ref):
  deg = dega_ref[...] + degb_ref[...] + 1.0
  p_ref[...] = lax.rsqrt(deg) * xp_ref[...]


def _prep(dega, degb, xp):
  vec = pl.BlockSpec((BN,), lambda i: (i,))
  return pl.pallas_call(
      _prep_body,
      grid=(N_PAD // BN,),
      in_specs=[vec, vec, vec],
      out_specs=vec,
      out_shape=jax.ShapeDtypeStruct((N_PAD,), jnp.float32),
  )(dega, degb, xp)


def _fin_body(dega_ref, degb_ref, ta_ref, tb_ref, p_ref, obs_ref,
              Wgz_ref, bgz_ref, Wlz_ref, blz_ref,
              Wgh_ref, bgh_ref, Wlh_ref, blh_ref, out_ref):
  Az = Wlz_ref[0:HID, :]
  uz = jnp.sum(Wgz_ref[0, :][:, None] * Az, axis=0)
  vz = jnp.sum(bgz_ref[:][:, None] * Az, axis=0) + blz_ref[...]
  Ah = Wlh_ref[0:HID, :]
  uh = jnp.sum(Wgh_ref[0, :][:, None] * Ah, axis=0)
  vh = jnp.sum(bgh_ref[:][:, None] * Ah, axis=0) + blh_ref[...]
  deg = dega_ref[...] + degb_ref[...] + 1.0
  dinv = lax.rsqrt(deg)
  t = ta_ref[...] + tb_ref[...]
  agg = dinv * t + dinv * p_ref[...]
  Z = jax.nn.sigmoid(agg[:, None] * uz[None, :] + vz[None, :])
  Ht = jnp.tanh(agg[:, None] * uh[None, :] + vh[None, :])
  H = (1.0 - Z) * Ht
  out_ref[...] = jnp.concatenate([obs_ref[:, 0:4], H], axis=1)[None]


def _fin(dega, degb, ta, tb, p, obs,
         Wg_z, bg_z, Wl_z, bl_z, Wg_h, bg_h, Wl_h, bl_h):
  def full(shape):
    return pl.BlockSpec(shape, lambda i, _s=shape: tuple(0 for _ in _s))
  vec = pl.BlockSpec((BN,), lambda i: (i,))
  return pl.pallas_call(
      _fin_body,
      grid=(N_PAD // BN,),
      in_specs=[
          vec, vec, vec, vec, vec,
          pl.BlockSpec((BN, 5), lambda i: (i, 0)),
          full((1, HID)), full((HID,)), full((2 * HID, HID)), full((HID,)),
          full((1, HID)), full((HID,)), full((2 * HID, HID)), full((HID,)),
      ],
      out_specs=pl.BlockSpec((1, BN, 4 + HID), lambda i: (0, i, 0)),
      out_shape=jax.ShapeDtypeStruct((1, N, 4 + HID), jnp.float32),
  )(dega, degb, ta, tb, p, obs,
    Wg_z, bg_z, Wl_z, bl_z, Wg_h, bg_h, Wl_h, bl_h)


def kernel(observations, edge_index, Wg_z, bg_z, Wl_z, bl_z, Wg_r, bg_r,
           Wl_r, bl_r, Wg_h, bg_h, Wl_h, bl_h):
  del Wg_r, bg_r, Wl_r, bl_r
  xp = jnp.pad(observations[:, 4], (0, N_PAD - N))
  dega, degb = _hist(edge_index)
  p = _prep(dega, degb, xp)
  ta, tb = _gs(edge_index, p)
  return _fin(dega, degb, ta, tb, p, observations,
              Wg_z, bg_z, Wl_z, bl_z, Wg_h, bg_h, Wl_h, bl_h)
